# baseline (speedup 1.0000x reference)
import functools
import math

import jax
import jax.numpy as jnp
import numpy as np
from jax.experimental import pallas as pl
from jax.experimental.pallas import tpu as pltpu


def _nt_dot(a, b):
    return jax.lax.dot_general(
        a, b, (((1,), (1,)), ((), ())), preferred_element_type=jnp.float32
    )


def _probsparse_kernel(q_ref, k_ref, v_ref, out_ref, idx_ref, qr_ref, *, L, D, u, u_pad, bk):
    q_mat = q_ref[0, 0]
    k_mat = k_ref[0, 0]
    v_mat = v_ref[0, 0]

    m_max = jnp.full((1, L), -jnp.inf, dtype=jnp.float32)
    for jb in range(L // bk):
        k_chunk = k_ref[0, 0, jb * bk:(jb + 1) * bk, :]
        s_t = _nt_dot(k_chunk, q_mat)
        m_max = jnp.maximum(m_max, jnp.max(s_t, axis=0, keepdims=True))
    k_sum = jnp.sum(k_mat, axis=0, keepdims=True)
    row_sum = _nt_dot(k_sum, q_mat)
    m_meas = m_max - row_sum * (1.0 / L)

    idx2d = jax.lax.broadcasted_iota(jnp.int32, (1, L), 1)
    m_cur = m_meas
    for t in range(u):
        mval = jnp.max(m_cur)
        sel = jnp.min(jnp.where(m_cur == mval, idx2d, L))
        idx_ref[t] = sel
        m_cur = jnp.where(idx2d == sel, -jnp.inf, m_cur)

    for t in range(u_pad):
        if t < u:
            qr_ref[t:t + 1, :] = q_ref[0, 0, pl.ds(idx_ref[t], 1), :]
        else:
            qr_ref[t:t + 1, :] = q_ref[0, 0, 0:1, :]
    q_red = qr_ref[...]
    scores = _nt_dot(q_red, k_mat) * (1.0 / math.sqrt(D))
    s_max = jnp.max(scores, axis=1, keepdims=True)
    s_exp = jnp.exp(scores - s_max)
    attn = s_exp / jnp.sum(s_exp, axis=1, keepdims=True)
    upd = jax.lax.dot_general(
        attn, v_mat, (((1,), (0,)), ((), ())), preferred_element_type=jnp.float32
    )

    v_mean = jnp.sum(v_mat, axis=0, keepdims=True) * (1.0 / L)
    out_ref[0, 0] = jnp.broadcast_to(v_mean, (L, D))
    for t in range(u):
        out_ref[0, 0, pl.ds(idx_ref[t], 1), :] = upd[t:t + 1, :]


def kernel(queries, keys, values, attn_mask):
    B, L, H, D = queries.shape
    u = 5 * int(np.ceil(np.log(L)))
    u = min(u, L)
    u_pad = ((u + 7) // 8) * 8
    bk = min(512, L)

    q_t = jnp.transpose(queries, (0, 2, 1, 3))
    k_t = jnp.transpose(keys, (0, 2, 1, 3))
    v_t = jnp.transpose(values, (0, 2, 1, 3))

    spec = pl.BlockSpec((1, 1, L, D), lambda b, h: (b, h, 0, 0))
    return pl.pallas_call(
        functools.partial(_probsparse_kernel, L=L, D=D, u=u, u_pad=u_pad, bk=bk),
        grid=(B, H),
        in_specs=[spec, spec, spec],
        out_specs=spec,
        out_shape=jax.ShapeDtypeStruct((B, H, L, D), jnp.float32),
        scratch_shapes=[
            pltpu.SMEM((u_pad,), jnp.int32),
            pltpu.VMEM((u_pad, D), jnp.float32),
        ],
    )(q_t, k_t, v_t)


# speedup vs baseline: 3.0810x; 3.0810x over previous
import functools
import math

import jax
import jax.numpy as jnp
import numpy as np
from jax.experimental import pallas as pl
from jax.experimental.pallas import tpu as pltpu


def _nt_dot(a, b):
    return jax.lax.dot_general(
        a, b, (((1,), (1,)), ((), ())), preferred_element_type=jnp.float32
    )


def _measure_kernel(q_ref, k_ref, m_ref, *, L, D, HP, bk):
    b = pl.program_id(0)
    hp = pl.program_id(1)
    r0 = 2 * (b * HP + hp)
    for hh in range(2):
        cols = slice(hh * D, (hh + 1) * D)
        q_h = q_ref[0][:, cols]
        m_max = jnp.full((1, L), -jnp.inf, dtype=jnp.float32)
        for jb in range(L // bk):
            k_chunk = k_ref[0, jb * bk:(jb + 1) * bk, cols]
            s_t = _nt_dot(k_chunk, q_h)
            m_max = jnp.maximum(m_max, jnp.max(s_t, axis=0, keepdims=True))
        k_sum = jnp.sum(k_ref[0][:, cols], axis=0, keepdims=True)
        row_sum = _nt_dot(k_sum, q_h)
        m_ref[pl.ds(r0 + hh, 1), :] = m_max - row_sum * (1.0 / L)


def _topk_kernel(m_ref, idx_ref, *, L, R, u, u_pad):
    m_cur = m_ref[...]
    idx2d = jax.lax.broadcasted_iota(jnp.int32, (R, L), 1)
    for t in range(u_pad):
        if t < u:
            rmax = jnp.max(m_cur, axis=1, keepdims=True)
            sel = jnp.min(
                jnp.where(m_cur == rmax, idx2d, L), axis=1, keepdims=True
            )
            idx_ref[:, t:t + 1] = sel
            m_cur = jnp.where(idx2d == sel, -jnp.inf, m_cur)
        else:
            idx_ref[:, t:t + 1] = jnp.zeros((R, 1), jnp.int32)


def _attend_kernel(idx_sref, q_ref, k_ref, v_ref, out_ref, qr_ref, *, L, D, HP, u, u_pad):
    b = pl.program_id(0)
    hp = pl.program_id(1)
    r0 = 2 * (b * HP + hp)
    scale = 1.0 / math.sqrt(D)
    for hh in range(2):
        cols = slice(hh * D, (hh + 1) * D)
        r = r0 + hh
        for t in range(u_pad):
            i = idx_sref[r, t] if t < u else 0
            qr_ref[t:t + 1, :] = q_ref[0, pl.ds(i, 1), cols]
        k_h = k_ref[0][:, cols]
        v_h = v_ref[0][:, cols]
        scores = _nt_dot(qr_ref[...], k_h) * scale
        s_max = jnp.max(scores, axis=1, keepdims=True)
        s_exp = jnp.exp(scores - s_max)
        attn = s_exp / jnp.sum(s_exp, axis=1, keepdims=True)
        upd = jax.lax.dot_general(
            attn, v_h, (((1,), (0,)), ((), ())), preferred_element_type=jnp.float32
        )
        v_mean = jnp.sum(v_h, axis=0, keepdims=True) * (1.0 / L)
        out_ref[0, hh] = jnp.broadcast_to(v_mean, (L, D))
        for t in range(u):
            out_ref[0, hh, pl.ds(idx_sref[r, t], 1), :] = upd[t:t + 1, :]


def kernel(queries, keys, values, attn_mask):
    B, L, H, D = queries.shape
    assert H % 2 == 0
    HP = H // 2
    R = B * H
    u = 5 * int(np.ceil(np.log(L)))
    u = min(u, L)
    u_pad = ((u + 7) // 8) * 8
    bk = min(512, L)

    q2 = jnp.reshape(queries, (B, L, H * D))
    k2 = jnp.reshape(keys, (B, L, H * D))
    v2 = jnp.reshape(values, (B, L, H * D))

    pair_spec = pl.BlockSpec((1, L, 2 * D), lambda b, hp: (b, 0, hp))

    m_all = pl.pallas_call(
        functools.partial(_measure_kernel, L=L, D=D, HP=HP, bk=bk),
        grid=(B, HP),
        in_specs=[pair_spec, pair_spec],
        out_specs=pl.BlockSpec((R, L), lambda b, hp: (0, 0)),
        out_shape=jax.ShapeDtypeStruct((R, L), jnp.float32),
    )(q2, k2)

    idx_all = pl.pallas_call(
        functools.partial(_topk_kernel, L=L, R=R, u=u, u_pad=u_pad),
        in_specs=[pl.BlockSpec((R, L), lambda: (0, 0))],
        out_specs=pl.BlockSpec((R, u_pad), lambda: (0, 0)),
        out_shape=jax.ShapeDtypeStruct((R, u_pad), jnp.int32),
    )(m_all)

    return pl.pallas_call(
        functools.partial(_attend_kernel, L=L, D=D, HP=HP, u=u, u_pad=u_pad),
        grid_spec=pltpu.PrefetchScalarGridSpec(
            num_scalar_prefetch=1,
            grid=(B, HP),
            in_specs=[
                pl.BlockSpec((1, L, 2 * D), lambda b, hp, idx: (b, 0, hp)),
                pl.BlockSpec((1, L, 2 * D), lambda b, hp, idx: (b, 0, hp)),
                pl.BlockSpec((1, L, 2 * D), lambda b, hp, idx: (b, 0, hp)),
            ],
            out_specs=pl.BlockSpec((1, 2, L, D), lambda b, hp, idx: (b, hp, 0, 0)),
            scratch_shapes=[pltpu.VMEM((u_pad, D), jnp.float32)],
        ),
        out_shape=jax.ShapeDtypeStruct((B, H, L, D), jnp.float32),
    )(idx_all, q2, k2, v2)
